# baseline (speedup 1.0000x reference)
import jax
import jax.numpy as jnp
from jax import lax
from jax.experimental import pallas as pl
from jax.experimental.pallas import tpu as pltpu

B = 4
C = 256
H = 56
N = H * H
NT = B * N
KNN = 4
TN = 448
NTILES = N // TN
F1 = 512
F2 = 256

_DEF = jax.lax.Precision.DEFAULT


def _k1_body(x_ref, idx_ref, dinv_ref, nf_ref):
    ti = pl.program_id(1)

    @pl.when(ti == 0)
    def _():
        x = x_ref[0]
        nrm = jnp.sqrt(jnp.sum(x * x, axis=1, keepdims=True))
        nf_ref[...] = x / jnp.maximum(nrm, 1e-12)
        dinv_ref[...] = jnp.zeros((1, N, 1), jnp.float32)

    rows = nf_ref[pl.ds(ti * TN, TN), :]
    nf = nf_ref[...]
    st = lax.dot_general(nf, rows, (((1,), (1,)), ((), ())),
                         preferred_element_type=jnp.float32,
                         precision=_DEF)
    rowid = lax.broadcasted_iota(jnp.int32, (N, TN), 0)
    histacc = jnp.zeros((N, TN), jnp.float32)
    args = []
    for t in range(KNN + 1):
        m = jnp.max(st, axis=0, keepdims=True)
        cand = jnp.where(st == m, rowid, N)
        arg = jnp.min(cand, axis=0)
        onehot = rowid == arg[None, :]
        if t >= 1:
            args.append(arg)
            histacc = histacc + onehot.astype(jnp.float32)
        if t < KNN:
            st = jnp.where(onehot, -jnp.inf, st)
    idx_ref[0, 0] = jnp.stack(args, axis=0)
    ones = jnp.ones((TN, 1), jnp.float32)
    hist = lax.dot_general(histacc, ones, (((1,), (0,)), ((), ())),
                           preferred_element_type=jnp.float32,
                           precision=_DEF)
    dinv_ref[...] += hist[None, :, :]

    @pl.when(ti == NTILES - 1)
    def _():
        dinv_ref[...] = lax.rsqrt(dinv_ref[...] + 2.0)


def _build_graph(xb):
    return pl.pallas_call(
        _k1_body,
        grid=(B, NTILES),
        in_specs=[pl.BlockSpec((1, N, C), lambda b, t: (b, 0, 0))],
        out_specs=[
            pl.BlockSpec((1, 1, KNN, TN), lambda b, t: (b, t, 0, 0)),
            pl.BlockSpec((1, N, 1), lambda b, t: (b, 0, 0)),
        ],
        out_shape=[
            jax.ShapeDtypeStruct((B, NTILES, KNN, TN), jnp.int32),
            jax.ShapeDtypeStruct((B, N, 1), jnp.float32),
        ],
        scratch_shapes=[pltpu.VMEM((N, C), jnp.float32)],
    )(xb)


def _k2a_body(x_ref, w_ref, d_ref, y_ref):
    xw = lax.dot_general(x_ref[...], w_ref[...], (((1,), (1,)), ((), ())),
                         preferred_element_type=jnp.float32, precision=_DEF)
    y_ref[...] = d_ref[...] * xw


def _xw_scaled(x, w, dinv_col):
    m, c = x.shape
    f = w.shape[0]
    return pl.pallas_call(
        _k2a_body,
        grid=(m // TN,),
        in_specs=[
            pl.BlockSpec((TN, c), lambda i: (i, 0)),
            pl.BlockSpec((f, c), lambda i: (0, 0)),
            pl.BlockSpec((TN, 1), lambda i: (i, 0)),
        ],
        out_specs=pl.BlockSpec((TN, f), lambda i: (i, 0)),
        out_shape=jax.ShapeDtypeStruct((m, f), jnp.float32),
    )(x, w, dinv_col)


def _adjt_tile(idx_ref, ti):
    rowid = lax.broadcasted_iota(jnp.int32, (TN, N), 0) + ti * TN
    at = jnp.zeros((TN, N), jnp.float32)
    for k in range(KNN):
        nbr_k = idx_ref[0, k, :]
        at += (nbr_k[None, :] == rowid).astype(jnp.float32)
    return at


def _k3_body(y_ref, idx_ref, d_ref, w_ref, b_ref, y2_ref):
    ti = pl.program_id(1)
    at = _adjt_tile(idx_ref, ti)
    acc = lax.dot_general(at, y_ref[0], (((1,), (0,)), ((), ())),
                          preferred_element_type=jnp.float32,
                          precision=_DEF)
    ytile = y_ref[0, pl.ds(ti * TN, TN), :]
    d = d_ref[0]
    h = jnp.maximum(d * (acc + 2.0 * ytile) + b_ref[...], 0.0)
    hw = lax.dot_general(h, w_ref[...], (((1,), (1,)), ((), ())),
                         preferred_element_type=jnp.float32, precision=_DEF)
    y2_ref[0] = d * hw


def _propagate_mid(y1, idx_t, dinv, w2, b1row):
    return pl.pallas_call(
        _k3_body,
        grid=(B, NTILES),
        in_specs=[
            pl.BlockSpec((1, N, F1), lambda b, t: (b, 0, 0)),
            pl.BlockSpec((1, KNN, N), lambda b, t: (b, 0, 0)),
            pl.BlockSpec((1, TN, 1), lambda b, t: (b, t, 0)),
            pl.BlockSpec((F2, F1), lambda b, t: (0, 0)),
            pl.BlockSpec((1, F1), lambda b, t: (0, 0)),
        ],
        out_specs=pl.BlockSpec((1, TN, F2), lambda b, t: (b, t, 0)),
        out_shape=jax.ShapeDtypeStruct((B, N, F2), jnp.float32),
    )(y1, idx_t, dinv, w2, b1row)


def _k4_body(y_ref, idx_ref, d_ref, b_ref, o_ref):
    ti = pl.program_id(1)
    at = _adjt_tile(idx_ref, ti)
    acc = lax.dot_general(at, y_ref[0], (((1,), (0,)), ((), ())),
                          preferred_element_type=jnp.float32,
                          precision=_DEF)
    ytile = y_ref[0, pl.ds(ti * TN, TN), :]
    d = d_ref[0]
    o_ref[0] = jnp.maximum(d * (acc + 2.0 * ytile) + b_ref[...], 0.0)


def _propagate_final(y2, idx_t, dinv, b2row):
    return pl.pallas_call(
        _k4_body,
        grid=(B, NTILES),
        in_specs=[
            pl.BlockSpec((1, N, F2), lambda b, t: (b, 0, 0)),
            pl.BlockSpec((1, KNN, N), lambda b, t: (b, 0, 0)),
            pl.BlockSpec((1, TN, 1), lambda b, t: (b, t, 0)),
            pl.BlockSpec((1, F2), lambda b, t: (0, 0)),
        ],
        out_specs=pl.BlockSpec((1, TN, F2), lambda b, t: (b, t, 0)),
        out_shape=jax.ShapeDtypeStruct((B, N, F2), jnp.float32),
    )(y2, idx_t, dinv, b2row)


def kernel(feature_maps, W1, b1, W2, b2):
    xb = jnp.transpose(feature_maps, (0, 2, 3, 1)).reshape(B, N, C)
    idxt, dinv_sub = _build_graph(xb)
    idx_t = jnp.transpose(idxt, (0, 2, 1, 3)).reshape(B, KNN, N)
    dinv_col = dinv_sub.reshape(NT, 1)
    x_flat = xb.reshape(NT, C)

    y1 = _xw_scaled(x_flat, W1, dinv_col).reshape(B, N, F1)
    y2 = _propagate_mid(y1, idx_t, dinv_sub, W2, b1.reshape(1, F1))
    out = _propagate_final(y2, idx_t, dinv_sub, b2.reshape(1, F2))
    return jnp.transpose(out.reshape(B, H, H, C), (0, 3, 1, 2))


# speedup vs baseline: 1.1124x; 1.1124x over previous
import jax
import jax.numpy as jnp
from jax import lax
from jax.experimental import pallas as pl
from jax.experimental.pallas import tpu as pltpu

B = 4
C = 256
H = 56
N = H * H
NT = B * N
KNN = 4
TN = 448
NTILES = N // TN
F1 = 512
F2 = 256

_DEF = jax.lax.Precision.DEFAULT


def _k1_body(x_ref, idx_ref, dinv_ref, nf_ref):
    ti = pl.program_id(1)

    @pl.when(ti == 0)
    def _():
        x = x_ref[0]
        nrm = jnp.sqrt(jnp.sum(x * x, axis=1, keepdims=True))
        nf_ref[...] = x / jnp.maximum(nrm, 1e-12)
        dinv_ref[...] = jnp.zeros((1, 1, N), jnp.float32)

    rows = nf_ref[pl.ds(ti * TN, TN), :]
    nf = nf_ref[...]
    s = lax.dot_general(rows, nf, (((1,), (1,)), ((), ())),
                        preferred_element_type=jnp.float32,
                        precision=_DEF)
    colf = lax.broadcasted_iota(jnp.int32, (TN, N), 1).astype(jnp.float32)
    hist = jnp.zeros((N,), jnp.float32)
    args = []
    for t in range(KNN + 1):
        m = jnp.max(s, axis=1, keepdims=True)
        cand = jnp.where(s == m, colf, float(N))
        arg = jnp.min(cand, axis=1)
        onehot = colf == arg[:, None]
        if t >= 1:
            args.append(arg.astype(jnp.int32))
            hist = hist + jnp.sum(onehot.astype(jnp.float32), axis=0)
        if t < KNN:
            s = jnp.where(onehot, -jnp.inf, s)
    idx_ref[0] = jnp.stack(args, axis=-1)
    dinv_ref[...] += hist[None, None, :]

    @pl.when(ti == NTILES - 1)
    def _():
        dinv_ref[...] = lax.rsqrt(dinv_ref[...] + 2.0)


def _build_graph(xb):
    return pl.pallas_call(
        _k1_body,
        grid=(B, NTILES),
        in_specs=[pl.BlockSpec((1, N, C), lambda b, t: (b, 0, 0))],
        out_specs=[
            pl.BlockSpec((1, TN, KNN), lambda b, t: (b, t, 0)),
            pl.BlockSpec((1, 1, N), lambda b, t: (b, 0, 0)),
        ],
        out_shape=[
            jax.ShapeDtypeStruct((B, N, KNN), jnp.int32),
            jax.ShapeDtypeStruct((B, 1, N), jnp.float32),
        ],
        scratch_shapes=[pltpu.VMEM((N, C), jnp.float32)],
    )(xb)


def _k2a_body(x_ref, w_ref, d_ref, y_ref):
    xw = lax.dot_general(x_ref[...], w_ref[...], (((1,), (1,)), ((), ())),
                         preferred_element_type=jnp.float32, precision=_DEF)
    y_ref[...] = d_ref[...] * xw


def _xw_scaled(x, w, dinv_col):
    m, c = x.shape
    f = w.shape[0]
    return pl.pallas_call(
        _k2a_body,
        grid=(m // TN,),
        in_specs=[
            pl.BlockSpec((TN, c), lambda i: (i, 0)),
            pl.BlockSpec((f, c), lambda i: (0, 0)),
            pl.BlockSpec((TN, 1), lambda i: (i, 0)),
        ],
        out_specs=pl.BlockSpec((TN, f), lambda i: (i, 0)),
        out_shape=jax.ShapeDtypeStruct((m, f), jnp.float32),
    )(x, w, dinv_col)


def _adjt_tile(idx_ref, ti):
    rowid = lax.broadcasted_iota(jnp.int32, (TN, N), 0) + ti * TN
    at = jnp.zeros((TN, N), jnp.float32)
    for k in range(KNN):
        nbr_k = idx_ref[0, k, :]
        at += (nbr_k[None, :] == rowid).astype(jnp.float32)
    return at.astype(jnp.bfloat16)


def _k3_body(y_ref, yb_ref, idx_ref, d_ref, w_ref, b_ref, y2_ref):
    ti = pl.program_id(1)
    at = _adjt_tile(idx_ref, ti)
    acc = lax.dot_general(at, yb_ref[0], (((1,), (0,)), ((), ())),
                          preferred_element_type=jnp.float32,
                          precision=_DEF)
    ytile = y_ref[0]
    d = d_ref[0]
    h = jnp.maximum(d * (acc + 2.0 * ytile) + b_ref[...], 0.0)
    hw = lax.dot_general(h, w_ref[...], (((1,), (1,)), ((), ())),
                         preferred_element_type=jnp.float32, precision=_DEF)
    y2_ref[0] = d * hw


def _propagate_mid(y1, y1b, idx_t, dinv, w2, b1row):
    return pl.pallas_call(
        _k3_body,
        grid=(B, NTILES),
        in_specs=[
            pl.BlockSpec((1, TN, F1), lambda b, t: (b, t, 0)),
            pl.BlockSpec((1, N, F1), lambda b, t: (b, 0, 0)),
            pl.BlockSpec((1, KNN, N), lambda b, t: (b, 0, 0)),
            pl.BlockSpec((1, TN, 1), lambda b, t: (b, t, 0)),
            pl.BlockSpec((F2, F1), lambda b, t: (0, 0)),
            pl.BlockSpec((1, F1), lambda b, t: (0, 0)),
        ],
        out_specs=pl.BlockSpec((1, TN, F2), lambda b, t: (b, t, 0)),
        out_shape=jax.ShapeDtypeStruct((B, N, F2), jnp.float32),
    )(y1, y1b, idx_t, dinv, w2, b1row)


def _k4_body(y_ref, yb_ref, idx_ref, d_ref, b_ref, o_ref):
    ti = pl.program_id(1)
    at = _adjt_tile(idx_ref, ti)
    acc = lax.dot_general(at, yb_ref[0], (((1,), (0,)), ((), ())),
                          preferred_element_type=jnp.float32,
                          precision=_DEF)
    ytile = y_ref[0]
    d = d_ref[0]
    o_ref[0] = jnp.maximum(d * (acc + 2.0 * ytile) + b_ref[...], 0.0)


def _propagate_final(y2, y2b, idx_t, dinv, b2row):
    return pl.pallas_call(
        _k4_body,
        grid=(B, NTILES),
        in_specs=[
            pl.BlockSpec((1, TN, F2), lambda b, t: (b, t, 0)),
            pl.BlockSpec((1, N, F2), lambda b, t: (b, 0, 0)),
            pl.BlockSpec((1, KNN, N), lambda b, t: (b, 0, 0)),
            pl.BlockSpec((1, TN, 1), lambda b, t: (b, t, 0)),
            pl.BlockSpec((1, F2), lambda b, t: (0, 0)),
        ],
        out_specs=pl.BlockSpec((1, TN, F2), lambda b, t: (b, t, 0)),
        out_shape=jax.ShapeDtypeStruct((B, N, F2), jnp.float32),
    )(y2, y2b, idx_t, dinv, b2row)


def kernel(feature_maps, W1, b1, W2, b2):
    xb = jnp.transpose(feature_maps, (0, 2, 3, 1)).reshape(B, N, C)
    idx, dinv = _build_graph(xb)
    idx_t = jnp.transpose(idx, (0, 2, 1))
    dinv_sub = jnp.transpose(dinv, (0, 2, 1))
    dinv_col = dinv_sub.reshape(NT, 1)
    x_flat = xb.reshape(NT, C)

    y1 = _xw_scaled(x_flat, W1, dinv_col).reshape(B, N, F1)
    y2 = _propagate_mid(y1, y1.astype(jnp.bfloat16), idx_t, dinv_sub,
                        W2, b1.reshape(1, F1))
    out = _propagate_final(y2, y2.astype(jnp.bfloat16), idx_t, dinv_sub,
                           b2.reshape(1, F2))
    return jnp.transpose(out.reshape(B, H, H, C), (0, 3, 1, 2))


# speedup vs baseline: 1.2077x; 1.0857x over previous
import jax
import jax.numpy as jnp
from jax import lax
from jax.experimental import pallas as pl
from jax.experimental.pallas import tpu as pltpu

B = 4
C = 256
H = 56
N = H * H
NT = B * N
KNN = 4
TN = 448
NTILES = N // TN
F1 = 512
F2 = 256

_DEF = jax.lax.Precision.DEFAULT


def _k1_body(x_ref, idx_ref, dinv_ref, nf_ref):
    ti = pl.program_id(1)

    @pl.when(ti == 0)
    def _():
        x = x_ref[0]
        nrm = jnp.sqrt(jnp.sum(x * x, axis=1, keepdims=True))
        nf_ref[...] = x / jnp.maximum(nrm, 1e-12)
        dinv_ref[...] = jnp.zeros((1, 1, N), jnp.float32)

    rows = nf_ref[pl.ds(ti * TN, TN), :]
    nf = nf_ref[...]
    s = lax.dot_general(rows, nf, (((1,), (1,)), ((), ())),
                        preferred_element_type=jnp.float32,
                        precision=_DEF)
    colf = lax.broadcasted_iota(jnp.int32, (TN, N), 1).astype(jnp.float32)
    args = []
    sel = None
    for t in range(KNN + 1):
        m = jnp.max(s, axis=1, keepdims=True)
        cand = jnp.where(s == m, colf, float(N))
        arg = jnp.min(cand, axis=1)
        onehot = colf == arg[:, None]
        if t >= 1:
            args.append(arg.astype(jnp.int32))
            sel = onehot if sel is None else (sel | onehot)
        if t < KNN:
            s = jnp.where(onehot, -jnp.inf, s)
    idx_ref[0] = jnp.stack(args, axis=-1)
    hist = jnp.sum(jnp.where(sel, 1.0, 0.0), axis=0)
    dinv_ref[...] += hist[None, None, :]

    @pl.when(ti == NTILES - 1)
    def _():
        dinv_ref[...] = lax.rsqrt(dinv_ref[...] + 2.0)


def _build_graph(xb):
    return pl.pallas_call(
        _k1_body,
        grid=(B, NTILES),
        in_specs=[pl.BlockSpec((1, N, C), lambda b, t: (b, 0, 0))],
        out_specs=[
            pl.BlockSpec((1, TN, KNN), lambda b, t: (b, t, 0)),
            pl.BlockSpec((1, 1, N), lambda b, t: (b, 0, 0)),
        ],
        out_shape=[
            jax.ShapeDtypeStruct((B, N, KNN), jnp.int32),
            jax.ShapeDtypeStruct((B, 1, N), jnp.float32),
        ],
        scratch_shapes=[pltpu.VMEM((N, C), jnp.float32)],
    )(xb)


def _k2a_body(x_ref, w_ref, d_ref, y_ref):
    xw = lax.dot_general(x_ref[...], w_ref[...], (((1,), (1,)), ((), ())),
                         preferred_element_type=jnp.float32, precision=_DEF)
    y_ref[...] = d_ref[...] * xw


def _xw_scaled(x, w, dinv_col):
    m, c = x.shape
    f = w.shape[0]
    return pl.pallas_call(
        _k2a_body,
        grid=(m // TN,),
        in_specs=[
            pl.BlockSpec((TN, c), lambda i: (i, 0)),
            pl.BlockSpec((f, c), lambda i: (0, 0)),
            pl.BlockSpec((TN, 1), lambda i: (i, 0)),
        ],
        out_specs=pl.BlockSpec((TN, f), lambda i: (i, 0)),
        out_shape=jax.ShapeDtypeStruct((m, f), jnp.float32),
    )(x, w, dinv_col)


def _adjt_tile(idx_ref, ti):
    rowid = (lax.broadcasted_iota(jnp.int32, (TN, N), 0)
             + ti * TN).astype(jnp.int16)
    hit = None
    for k in range(KNN):
        nbr_k = idx_ref[0, k, :].astype(jnp.int16)
        e = nbr_k[None, :] == rowid
        hit = e if hit is None else (hit | e)
    return jnp.where(hit, jnp.bfloat16(1.0), jnp.bfloat16(0.0))


def _k3_body(y_ref, yb_ref, idx_ref, d_ref, w_ref, b_ref, y2_ref):
    ti = pl.program_id(1)
    at = _adjt_tile(idx_ref, ti)
    acc = lax.dot_general(at, yb_ref[0], (((1,), (0,)), ((), ())),
                          preferred_element_type=jnp.float32,
                          precision=_DEF)
    ytile = y_ref[0]
    d = d_ref[0]
    h = jnp.maximum(d * (acc + 2.0 * ytile) + b_ref[...], 0.0)
    hw = lax.dot_general(h, w_ref[...], (((1,), (1,)), ((), ())),
                         preferred_element_type=jnp.float32, precision=_DEF)
    y2_ref[0] = d * hw


def _propagate_mid(y1, y1b, idx_t, dinv, w2, b1row):
    return pl.pallas_call(
        _k3_body,
        grid=(B, NTILES),
        in_specs=[
            pl.BlockSpec((1, TN, F1), lambda b, t: (b, t, 0)),
            pl.BlockSpec((1, N, F1), lambda b, t: (b, 0, 0)),
            pl.BlockSpec((1, KNN, N), lambda b, t: (b, 0, 0)),
            pl.BlockSpec((1, TN, 1), lambda b, t: (b, t, 0)),
            pl.BlockSpec((F2, F1), lambda b, t: (0, 0)),
            pl.BlockSpec((1, F1), lambda b, t: (0, 0)),
        ],
        out_specs=pl.BlockSpec((1, TN, F2), lambda b, t: (b, t, 0)),
        out_shape=jax.ShapeDtypeStruct((B, N, F2), jnp.float32),
    )(y1, y1b, idx_t, dinv, w2, b1row)


def _k4_body(y_ref, yb_ref, idx_ref, d_ref, b_ref, o_ref):
    ti = pl.program_id(1)
    at = _adjt_tile(idx_ref, ti)
    acc = lax.dot_general(at, yb_ref[0], (((1,), (0,)), ((), ())),
                          preferred_element_type=jnp.float32,
                          precision=_DEF)
    ytile = y_ref[0]
    d = d_ref[0]
    o_ref[0] = jnp.maximum(d * (acc + 2.0 * ytile) + b_ref[...], 0.0)


def _propagate_final(y2, y2b, idx_t, dinv, b2row):
    return pl.pallas_call(
        _k4_body,
        grid=(B, NTILES),
        in_specs=[
            pl.BlockSpec((1, TN, F2), lambda b, t: (b, t, 0)),
            pl.BlockSpec((1, N, F2), lambda b, t: (b, 0, 0)),
            pl.BlockSpec((1, KNN, N), lambda b, t: (b, 0, 0)),
            pl.BlockSpec((1, TN, 1), lambda b, t: (b, t, 0)),
            pl.BlockSpec((1, F2), lambda b, t: (0, 0)),
        ],
        out_specs=pl.BlockSpec((1, TN, F2), lambda b, t: (b, t, 0)),
        out_shape=jax.ShapeDtypeStruct((B, N, F2), jnp.float32),
    )(y2, y2b, idx_t, dinv, b2row)


def kernel(feature_maps, W1, b1, W2, b2):
    xb = jnp.transpose(feature_maps, (0, 2, 3, 1)).reshape(B, N, C)
    idx, dinv = _build_graph(xb)
    idx_t = jnp.transpose(idx, (0, 2, 1))
    dinv_sub = jnp.transpose(dinv, (0, 2, 1))
    dinv_col = dinv_sub.reshape(NT, 1)
    x_flat = xb.reshape(NT, C)

    y1 = _xw_scaled(x_flat, W1, dinv_col).reshape(B, N, F1)
    y2 = _propagate_mid(y1, y1.astype(jnp.bfloat16), idx_t, dinv_sub,
                        W2, b1.reshape(1, F1))
    out = _propagate_final(y2, y2.astype(jnp.bfloat16), idx_t, dinv_sub,
                           b2.reshape(1, F2))
    return jnp.transpose(out.reshape(B, H, H, C), (0, 3, 1, 2))


# speedup vs baseline: 1.2690x; 1.0507x over previous
import jax
import jax.numpy as jnp
from jax import lax
from jax.experimental import pallas as pl
from jax.experimental.pallas import tpu as pltpu

B = 4
C = 256
H = 56
N = H * H
NT = B * N
KNN = 4
TN = 448
NTILES = N // TN
F1 = 512
F2 = 256

_DEF = jax.lax.Precision.DEFAULT


def _k1_body(x_ref, idx_ref, dinv_ref, nf_ref):
    ti = pl.program_id(1)

    @pl.when(ti == 0)
    def _():
        x = x_ref[0]
        nrm = jnp.sqrt(jnp.sum(x * x, axis=1, keepdims=True))
        nf_ref[...] = x / jnp.maximum(nrm, 1e-12)
        dinv_ref[...] = jnp.zeros((1, 1, N), jnp.float32)

    rows = nf_ref[pl.ds(ti * TN, TN), :]
    nf = nf_ref[...]
    s = lax.dot_general(rows, nf, (((1,), (1,)), ((), ())),
                        preferred_element_type=jnp.float32,
                        precision=_DEF)
    coli = lax.broadcasted_iota(jnp.int32, (TN, N), 1)
    args = []
    sel = None
    for t in range(KNN + 1):
        arg = jnp.argmax(s, axis=1).astype(jnp.int32)
        onehot = coli == arg[:, None]
        if t >= 1:
            args.append(arg)
            sel = onehot if sel is None else (sel | onehot)
        if t < KNN:
            s = jnp.where(onehot, -jnp.inf, s)
    idx_ref[0] = jnp.stack(args, axis=-1)
    hist = jnp.sum(jnp.where(sel, 1.0, 0.0), axis=0)
    dinv_ref[...] += hist[None, None, :]

    @pl.when(ti == NTILES - 1)
    def _():
        dinv_ref[...] = lax.rsqrt(dinv_ref[...] + 2.0)


def _build_graph(xb):
    return pl.pallas_call(
        _k1_body,
        grid=(B, NTILES),
        in_specs=[pl.BlockSpec((1, N, C), lambda b, t: (b, 0, 0))],
        out_specs=[
            pl.BlockSpec((1, TN, KNN), lambda b, t: (b, t, 0)),
            pl.BlockSpec((1, 1, N), lambda b, t: (b, 0, 0)),
        ],
        out_shape=[
            jax.ShapeDtypeStruct((B, N, KNN), jnp.int32),
            jax.ShapeDtypeStruct((B, 1, N), jnp.float32),
        ],
        scratch_shapes=[pltpu.VMEM((N, C), jnp.float32)],
    )(xb)


def _k2a_body(x_ref, w_ref, d_ref, y_ref):
    xw = lax.dot_general(x_ref[...], w_ref[...], (((1,), (1,)), ((), ())),
                         preferred_element_type=jnp.float32, precision=_DEF)
    y_ref[...] = d_ref[...] * xw


def _xw_scaled(x, w, dinv_col):
    m, c = x.shape
    f = w.shape[0]
    return pl.pallas_call(
        _k2a_body,
        grid=(m // TN,),
        in_specs=[
            pl.BlockSpec((TN, c), lambda i: (i, 0)),
            pl.BlockSpec((f, c), lambda i: (0, 0)),
            pl.BlockSpec((TN, 1), lambda i: (i, 0)),
        ],
        out_specs=pl.BlockSpec((TN, f), lambda i: (i, 0)),
        out_shape=jax.ShapeDtypeStruct((m, f), jnp.float32),
    )(x, w, dinv_col)


def _adjt_tile(idx_ref, ti):
    rowid = (lax.broadcasted_iota(jnp.int32, (TN, N), 0)
             + ti * TN).astype(jnp.int16)
    hit = None
    for k in range(KNN):
        nbr_k = idx_ref[0, k, :].astype(jnp.int16)
        e = nbr_k[None, :] == rowid
        hit = e if hit is None else (hit | e)
    return jnp.where(hit, jnp.bfloat16(1.0), jnp.bfloat16(0.0))


def _k3_body(y_ref, yb_ref, idx_ref, d_ref, w_ref, b_ref, y2_ref):
    ti = pl.program_id(1)
    at = _adjt_tile(idx_ref, ti)
    acc = lax.dot_general(at, yb_ref[0], (((1,), (0,)), ((), ())),
                          preferred_element_type=jnp.float32,
                          precision=_DEF)
    ytile = y_ref[0]
    d = d_ref[0]
    h = jnp.maximum(d * (acc + 2.0 * ytile) + b_ref[...], 0.0)
    hw = lax.dot_general(h, w_ref[...], (((1,), (1,)), ((), ())),
                         preferred_element_type=jnp.float32, precision=_DEF)
    y2_ref[0] = d * hw


def _propagate_mid(y1, y1b, idx_t, dinv, w2, b1row):
    return pl.pallas_call(
        _k3_body,
        grid=(B, NTILES),
        in_specs=[
            pl.BlockSpec((1, TN, F1), lambda b, t: (b, t, 0)),
            pl.BlockSpec((1, N, F1), lambda b, t: (b, 0, 0)),
            pl.BlockSpec((1, KNN, N), lambda b, t: (b, 0, 0)),
            pl.BlockSpec((1, TN, 1), lambda b, t: (b, t, 0)),
            pl.BlockSpec((F2, F1), lambda b, t: (0, 0)),
            pl.BlockSpec((1, F1), lambda b, t: (0, 0)),
        ],
        out_specs=pl.BlockSpec((1, TN, F2), lambda b, t: (b, t, 0)),
        out_shape=jax.ShapeDtypeStruct((B, N, F2), jnp.float32),
    )(y1, y1b, idx_t, dinv, w2, b1row)


def _k4_body(y_ref, yb_ref, idx_ref, d_ref, b_ref, o_ref):
    ti = pl.program_id(1)
    at = _adjt_tile(idx_ref, ti)
    acc = lax.dot_general(at, yb_ref[0], (((1,), (0,)), ((), ())),
                          preferred_element_type=jnp.float32,
                          precision=_DEF)
    ytile = y_ref[0]
    d = d_ref[0]
    o_ref[0] = jnp.maximum(d * (acc + 2.0 * ytile) + b_ref[...], 0.0)


def _propagate_final(y2, y2b, idx_t, dinv, b2row):
    return pl.pallas_call(
        _k4_body,
        grid=(B, NTILES),
        in_specs=[
            pl.BlockSpec((1, TN, F2), lambda b, t: (b, t, 0)),
            pl.BlockSpec((1, N, F2), lambda b, t: (b, 0, 0)),
            pl.BlockSpec((1, KNN, N), lambda b, t: (b, 0, 0)),
            pl.BlockSpec((1, TN, 1), lambda b, t: (b, t, 0)),
            pl.BlockSpec((1, F2), lambda b, t: (0, 0)),
        ],
        out_specs=pl.BlockSpec((1, TN, F2), lambda b, t: (b, t, 0)),
        out_shape=jax.ShapeDtypeStruct((B, N, F2), jnp.float32),
    )(y2, y2b, idx_t, dinv, b2row)


def kernel(feature_maps, W1, b1, W2, b2):
    xb = jnp.transpose(feature_maps, (0, 2, 3, 1)).reshape(B, N, C)
    idx, dinv = _build_graph(xb)
    idx_t = jnp.transpose(idx, (0, 2, 1))
    dinv_sub = jnp.transpose(dinv, (0, 2, 1))
    dinv_col = dinv_sub.reshape(NT, 1)
    x_flat = xb.reshape(NT, C)

    y1 = _xw_scaled(x_flat, W1, dinv_col).reshape(B, N, F1)
    y2 = _propagate_mid(y1, y1.astype(jnp.bfloat16), idx_t, dinv_sub,
                        W2, b1.reshape(1, F1))
    out = _propagate_final(y2, y2.astype(jnp.bfloat16), idx_t, dinv_sub,
                           b2.reshape(1, F2))
    return jnp.transpose(out.reshape(B, H, H, C), (0, 3, 1, 2))


# speedup vs baseline: 1.4518x; 1.1440x over previous
import jax
import jax.numpy as jnp
from jax import lax
from jax.experimental import pallas as pl
from jax.experimental.pallas import tpu as pltpu

B = 4
C = 256
H = 56
N = H * H
NT = B * N
KNN = 4
TN = 448
NTILES = N // TN
F1 = 512
F2 = 256

_DEF = jax.lax.Precision.DEFAULT


def _k1_body(x_ref, idx_ref, dinv_ref, nf_ref):
    ti = pl.program_id(1)

    @pl.when(ti == 0)
    def _():
        x = x_ref[0]
        nrm = jnp.sqrt(jnp.sum(x * x, axis=1, keepdims=True))
        nf_ref[...] = x / jnp.maximum(nrm, 1e-12)
        dinv_ref[...] = jnp.zeros((1, 1, N), jnp.float32)

    rows = nf_ref[pl.ds(ti * TN, TN), :]
    nf = nf_ref[...]
    s = lax.dot_general(rows, nf, (((1,), (1,)), ((), ())),
                        preferred_element_type=jnp.float32,
                        precision=_DEF)
    coli = lax.broadcasted_iota(jnp.int32, (TN, N), 1)
    args = []
    sel = None
    for t in range(KNN + 1):
        arg = jnp.argmax(s, axis=1).astype(jnp.int32)
        onehot = coli == arg[:, None]
        if t >= 1:
            args.append(arg)
            sel = onehot if sel is None else (sel | onehot)
        if t < KNN:
            s = jnp.where(onehot, -jnp.inf, s)
    idx_ref[0] = jnp.stack(args, axis=-1)
    hist = jnp.sum(jnp.where(sel, 1.0, 0.0), axis=0)
    dinv_ref[...] += hist[None, None, :]

    @pl.when(ti == NTILES - 1)
    def _():
        dinv_ref[...] = lax.rsqrt(dinv_ref[...] + 2.0)


def _build_graph(xb):
    return pl.pallas_call(
        _k1_body,
        grid=(B, NTILES),
        in_specs=[pl.BlockSpec((1, N, C), lambda b, t: (b, 0, 0))],
        out_specs=[
            pl.BlockSpec((1, TN, KNN), lambda b, t: (b, t, 0)),
            pl.BlockSpec((1, 1, N), lambda b, t: (b, 0, 0)),
        ],
        out_shape=[
            jax.ShapeDtypeStruct((B, N, KNN), jnp.int32),
            jax.ShapeDtypeStruct((B, 1, N), jnp.float32),
        ],
        scratch_shapes=[pltpu.VMEM((N, C), jnp.float32)],
    )(xb)


def _adjt_tile(idx_ref, ti):
    rowid = (lax.broadcasted_iota(jnp.int32, (TN, N), 0)
             + ti * TN).astype(jnp.int16)
    hit = None
    for k in range(KNN):
        nbr_k = idx_ref[0, k, :].astype(jnp.int16)
        e = nbr_k[None, :] == rowid
        hit = e if hit is None else (hit | e)
    return jnp.where(hit, jnp.bfloat16(1.0), jnp.bfloat16(0.0))


def _k3_body(x_ref, idx_ref, d_ref, w1_ref, w2_ref, b_ref, y2_ref,
             y1_ref, y1b_ref):
    ti = pl.program_id(1)

    @pl.when(ti == 0)
    def _():
        xw = lax.dot_general(x_ref[0], w1_ref[...], (((1,), (1,)), ((), ())),
                             preferred_element_type=jnp.float32,
                             precision=_DEF)
        y1 = d_ref[0] * xw
        y1_ref[...] = y1
        y1b_ref[...] = y1.astype(jnp.bfloat16)

    at = _adjt_tile(idx_ref, ti)
    acc = lax.dot_general(at, y1b_ref[...], (((1,), (0,)), ((), ())),
                          preferred_element_type=jnp.float32,
                          precision=_DEF)
    ytile = y1_ref[pl.ds(ti * TN, TN), :]
    d = d_ref[0, pl.ds(ti * TN, TN), :]
    h = jnp.maximum(d * (acc + 2.0 * ytile) + b_ref[...], 0.0)
    hw = lax.dot_general(h, w2_ref[...], (((1,), (1,)), ((), ())),
                         preferred_element_type=jnp.float32, precision=_DEF)
    y2_ref[0] = d * hw


def _propagate_mid(xb, idx_t, dinv, w1, w2, b1row):
    return pl.pallas_call(
        _k3_body,
        grid=(B, NTILES),
        in_specs=[
            pl.BlockSpec((1, N, C), lambda b, t: (b, 0, 0)),
            pl.BlockSpec((1, KNN, N), lambda b, t: (b, 0, 0)),
            pl.BlockSpec((1, N, 1), lambda b, t: (b, 0, 0)),
            pl.BlockSpec((F1, C), lambda b, t: (0, 0)),
            pl.BlockSpec((F2, F1), lambda b, t: (0, 0)),
            pl.BlockSpec((1, F1), lambda b, t: (0, 0)),
        ],
        out_specs=pl.BlockSpec((1, TN, F2), lambda b, t: (b, t, 0)),
        out_shape=jax.ShapeDtypeStruct((B, N, F2), jnp.float32),
        scratch_shapes=[pltpu.VMEM((N, F1), jnp.float32),
                        pltpu.VMEM((N, F1), jnp.bfloat16)],
    )(xb, idx_t, dinv, w1, w2, b1row)


def _k4_body(y_ref, idx_ref, d_ref, b_ref, o_ref, y2b_ref):
    ti = pl.program_id(1)

    @pl.when(ti == 0)
    def _():
        y2b_ref[...] = y_ref[0].astype(jnp.bfloat16)

    at = _adjt_tile(idx_ref, ti)
    acc = lax.dot_general(at, y2b_ref[...], (((1,), (0,)), ((), ())),
                          preferred_element_type=jnp.float32,
                          precision=_DEF)
    ytile = y_ref[0, pl.ds(ti * TN, TN), :]
    d = d_ref[0, pl.ds(ti * TN, TN), :]
    o_ref[0] = jnp.maximum(d * (acc + 2.0 * ytile) + b_ref[...], 0.0)


def _propagate_final(y2, idx_t, dinv, b2row):
    return pl.pallas_call(
        _k4_body,
        grid=(B, NTILES),
        in_specs=[
            pl.BlockSpec((1, N, F2), lambda b, t: (b, 0, 0)),
            pl.BlockSpec((1, KNN, N), lambda b, t: (b, 0, 0)),
            pl.BlockSpec((1, N, 1), lambda b, t: (b, 0, 0)),
            pl.BlockSpec((1, F2), lambda b, t: (0, 0)),
        ],
        out_specs=pl.BlockSpec((1, TN, F2), lambda b, t: (b, t, 0)),
        out_shape=jax.ShapeDtypeStruct((B, N, F2), jnp.float32),
        scratch_shapes=[pltpu.VMEM((N, F2), jnp.bfloat16)],
    )(y2, idx_t, dinv, b2row)


def kernel(feature_maps, W1, b1, W2, b2):
    xb = jnp.transpose(feature_maps, (0, 2, 3, 1)).reshape(B, N, C)
    idx, dinv = _build_graph(xb)
    idx_t = jnp.transpose(idx, (0, 2, 1))
    dinv_sub = jnp.transpose(dinv, (0, 2, 1))

    y2 = _propagate_mid(xb, idx_t, dinv_sub, W1, W2, b1.reshape(1, F1))
    out = _propagate_final(y2, idx_t, dinv_sub, b2.reshape(1, F2))
    return jnp.transpose(out.reshape(B, H, H, C), (0, 3, 1, 2))
